# baseline (speedup 1.0000x reference)
import jax
import jax.numpy as jnp
from jax import lax
from jax.experimental import pallas as pl
from jax.experimental.pallas import tpu as pltpu
from jax.experimental.pallas import tpu_sc as plsc

_CLS = 1
_B = 16
_L = 4096
_W = _L + 1
_NTOT = _B * _W
_NIN = _B * _L
_LANES = 16
_PIECE = 2048
_CHUNKS = _PIECE // _LANES
_TAIL = _NTOT - 32 * _PIECE
_VIN = _PIECE + 32


def _work(wid, values_hbm, lengths_hbm, out_hbm, newlen_hbm, lens_v, vin,
          vpiece, newlen_v):
    iota = lax.iota(jnp.int32, _LANES)

    o0 = wid * _PIECE
    r0 = o0 // _W
    a_raw = o0 - r0 - 1
    a0 = jnp.clip((a_raw // 8) * 8, 0, _NIN - _VIN)
    a0 = pl.multiple_of(a0, 8)

    pltpu.sync_copy(lengths_hbm, lens_v)
    pltpu.sync_copy(values_hbm.at[pl.ds(a0, _VIN)], vin)

    lens = lens_v[...]

    def splat(idx):
        return lax.gather(
            lens, jnp.full((_LANES, 1), idx, jnp.int32),
            dimension_numbers=lax.GatherDimensionNumbers(
                offset_dims=(), collapsed_slice_dims=(0,), start_index_map=(0,)),
            slice_sizes=(1,),
            mode=lax.GatherScatterMode.PROMISE_IN_BOUNDS,
        )

    rb = (r0 + 1) * _W
    len0 = splat(r0)
    len1 = splat(jnp.minimum(r0 + 1, _B - 1))
    rbv = iota * 0 + rb
    z0 = r0 * _W + len0 + 1
    z1 = rb + len1 + 1
    c1 = -r0 - 1 - a0

    def chunk(i, kk):
        m2 = kk >= rbv
        sl = jnp.maximum(kk + c1 - m2.astype(jnp.int32), 0)
        g = plsc.load_gather(vin, [sl])
        valid = (kk < z0) | (m2 & (kk < z1))
        vpiece[pl.ds(i * _LANES, _LANES)] = jnp.where(valid, g, 0)
        return kk + _LANES

    lax.fori_loop(0, _CHUNKS + 1, chunk, o0 + iota, unroll=4)

    cls_vec = jnp.full((_LANES,), _CLS, jnp.int32)
    lane0 = iota == 0

    @pl.when(o0 == 0)
    def _():
        plsc.store_scatter(vpiece, [iota * 0], cls_vec, mask=lane0)

    rb_local = rb - o0

    @pl.when(rb_local < _PIECE + _TAIL)
    def _():
        plsc.store_scatter(vpiece, [iota * 0 + rb_local], cls_vec, mask=lane0)

    last = wid == 31

    @pl.when(last)
    def _():
        pltpu.sync_copy(vpiece, out_hbm.at[pl.ds(o0, _PIECE + _TAIL)])

    @pl.when(jnp.logical_not(last))
    def _():
        pltpu.sync_copy(vpiece.at[pl.ds(0, _PIECE)],
                        out_hbm.at[pl.ds(o0, _PIECE)])

    @pl.when(wid == 0)
    def _():
        newlen_v[...] = lens + 1
        pltpu.sync_copy(newlen_v, newlen_hbm)


def _body(values_hbm, lengths_hbm, out_hbm, newlen_hbm, lens_v, vin, vpiece,
          newlen_v):
    s = lax.axis_index("s")
    for rep in range(2):
        _work(rep * _LANES + s, values_hbm, lengths_hbm, out_hbm, newlen_hbm,
              lens_v, vin, vpiece, newlen_v)


@jax.jit
def _prepend_cls(values_flat, lengths):
    mesh = plsc.VectorSubcoreMesh(core_axis_name="c", subcore_axis_name="s",
                                  num_cores=1)
    f = pl.kernel(
        _body,
        out_type=(
            jax.ShapeDtypeStruct((_NTOT,), jnp.int32),
            jax.ShapeDtypeStruct((_B,), jnp.int32),
        ),
        mesh=mesh,
        compiler_params=pltpu.CompilerParams(needs_layout_passes=False,
                                             skip_device_barrier=True),
        scratch_types=[
            pltpu.VMEM((_LANES,), jnp.int32),
            pltpu.VMEM((_VIN,), jnp.int32),
            pltpu.VMEM((_PIECE + _TAIL,), jnp.int32),
            pltpu.VMEM((_LANES,), jnp.int32),
        ],
    )
    return f(values_flat, lengths)


def kernel(values, lengths):
    v = values.astype(jnp.int32).reshape(_NIN)
    l = lengths.astype(jnp.int32)
    out_flat, new_lengths = _prepend_cls(v, l)
    out = out_flat.reshape(_B, _W).astype(values.dtype)
    return out, new_lengths.astype(lengths.dtype)


# speedup vs baseline: 1.0556x; 1.0556x over previous
import jax
import jax.numpy as jnp
from jax import lax
from jax.experimental import pallas as pl
from jax.experimental.pallas import tpu as pltpu
from jax.experimental.pallas import tpu_sc as plsc

_CLS = 1
_B = 16
_L = 4096
_W = _L + 1
_NTOT = _B * _W
_NIN = _B * _L
_LANES = 16
_PIECE = 2048
_CHUNKS = _PIECE // _LANES
_TAIL = _NTOT - 32 * _PIECE
_VIN = _PIECE + 32


def _body(values_hbm, lengths_hbm, out_hbm, newlen_hbm, lens_v, vin, vpiece,
          newlen_v, sem):
    c = lax.axis_index("c")
    s = lax.axis_index("s")
    iota = lax.iota(jnp.int32, _LANES)

    o0 = (c * _LANES + s) * _PIECE
    r0 = o0 // _W
    a_raw = o0 - r0 - 1
    a0 = jnp.clip((a_raw // 8) * 8, 0, _NIN - _VIN)
    a0 = pl.multiple_of(a0, 8)

    cp1 = pltpu.make_async_copy(lengths_hbm, lens_v, sem)
    cp1.start()
    cp2 = pltpu.make_async_copy(values_hbm.at[pl.ds(a0, _VIN)], vin, sem)
    cp2.start()
    cp1.wait()
    cp2.wait()

    lens = lens_v[...]

    def splat(idx):
        return lax.gather(
            lens, jnp.full((_LANES, 1), idx, jnp.int32),
            dimension_numbers=lax.GatherDimensionNumbers(
                offset_dims=(), collapsed_slice_dims=(0,), start_index_map=(0,)),
            slice_sizes=(1,),
            mode=lax.GatherScatterMode.PROMISE_IN_BOUNDS,
        )

    rb = (r0 + 1) * _W
    len0 = splat(r0)
    len1 = splat(jnp.minimum(r0 + 1, _B - 1))
    rbv = iota * 0 + rb
    z0 = r0 * _W + len0 + 1
    z1 = rb + len1 + 1
    c1 = -r0 - 1 - a0

    kk0 = o0 + iota

    @plsc.parallel_loop(0, _CHUNKS + 1, unroll=4)
    def _(i):
        kk = kk0 + i * _LANES
        m2 = kk >= rbv
        sl = jnp.maximum(kk + c1 - m2.astype(jnp.int32), 0)
        g = plsc.load_gather(vin, [sl])
        valid = (kk < z0) | (m2 & (kk < z1))
        vpiece[pl.ds(i * _LANES, _LANES)] = jnp.where(valid, g, 0)

    cls_vec = jnp.full((_LANES,), _CLS, jnp.int32)
    lane0 = iota == 0

    @pl.when(o0 == 0)
    def _():
        plsc.store_scatter(vpiece, [iota * 0], cls_vec, mask=lane0)

    rb_local = rb - o0

    @pl.when(rb_local < _PIECE + _TAIL)
    def _():
        plsc.store_scatter(vpiece, [iota * 0 + rb_local], cls_vec, mask=lane0)

    last = jnp.logical_and(c == 1, s == _LANES - 1)

    @pl.when(last)
    def _():
        pltpu.sync_copy(vpiece, out_hbm.at[pl.ds(o0, _PIECE + _TAIL)])

    @pl.when(jnp.logical_not(last))
    def _():
        pltpu.sync_copy(vpiece.at[pl.ds(0, _PIECE)],
                        out_hbm.at[pl.ds(o0, _PIECE)])

    @pl.when(jnp.logical_and(s == 0, c == 0))
    def _():
        newlen_v[...] = lens + 1
        pltpu.sync_copy(newlen_v, newlen_hbm)


@jax.jit
def _prepend_cls(values_flat, lengths):
    mesh = plsc.VectorSubcoreMesh(core_axis_name="c", subcore_axis_name="s")
    f = pl.kernel(
        _body,
        out_type=(
            jax.ShapeDtypeStruct((_NTOT,), jnp.int32),
            jax.ShapeDtypeStruct((_B,), jnp.int32),
        ),
        mesh=mesh,
        compiler_params=pltpu.CompilerParams(needs_layout_passes=False,
                                             skip_device_barrier=True),
        scratch_types=[
            pltpu.VMEM((_LANES,), jnp.int32),
            pltpu.VMEM((_VIN,), jnp.int32),
            pltpu.VMEM((_PIECE + _TAIL,), jnp.int32),
            pltpu.VMEM((_LANES,), jnp.int32),
            pltpu.SemaphoreType.DMA,
        ],
    )
    return f(values_flat, lengths)


def kernel(values, lengths):
    v = values.astype(jnp.int32).reshape(_NIN)
    l = lengths.astype(jnp.int32)
    out_flat, new_lengths = _prepend_cls(v, l)
    out = out_flat.reshape(_B, _W).astype(values.dtype)
    return out, new_lengths.astype(lengths.dtype)
